# Initial kernel scaffold; baseline (speedup 1.0000x reference)
import functools

import jax
import jax.numpy as jnp
from jax import lax
from jax.experimental import pallas as pl
from jax.experimental.pallas import tpu as pltpu
from jax.experimental.pallas import tpu_sc as plsc

N = 10000
E = 320000
DF = 128
DE = 16
H = 64
L = 3
G = 64

NC = 2
NS = 16
NW = NC * NS
CHUNK = 128
EPW = 10240
EP = NW * EPW
NCHUNKS = EPW // CHUNK
NP = 10016
RPT = NP // NS
_ROW_CHUNKS = ((0, 128), (128, 128), (256, 128), (384, 128), (512, 114))


def _sc_body(layer, h_hbm, src2_hbm, dst2_hbm, eemb_hbm, out_hbm,
             src_v, dst_v, rows_v, emb_v, agg_s, sem):
    c = lax.axis_index("c")
    s = lax.axis_index("s")
    w = s * NC + c

    def _zrow(i, carry):
        for k in range(4):
            rows_v[i, pl.ds(16 * k, 16)] = jnp.zeros((16,), jnp.float32)
        return carry
    lax.fori_loop(0, CHUNK, _zrow, 0, unroll=4)
    base_r = s * RPT
    for off, nr in _ROW_CHUNKS:
        pltpu.sync_copy(rows_v.at[pl.ds(0, nr)], agg_s.at[pl.ds(base_r + off, nr)])
    plsc.subcore_barrier()

    def _chunk(ci, carry):
        r = w * NCHUNKS + ci
        e0 = r * CHUNK
        pltpu.sync_copy(src2_hbm.at[r], src_v.at[0])
        pltpu.sync_copy(dst2_hbm.at[r], dst_v.at[0])
        pltpu.async_copy(h_hbm.at[src_v.at[0]], rows_v, sem).wait()
        pltpu.sync_copy(eemb_hbm.at[layer, pl.ds(e0, CHUNK)], emb_v)

        def _vrow(i, cy):
            for k in range(4):
                sl = pl.ds(16 * k, 16)
                rows_v[i, sl] = jnp.maximum(rows_v[i, sl] + emb_v[i, sl], 0.0)
            return cy
        lax.fori_loop(0, CHUNK, _vrow, 0, unroll=2)
        pltpu.sync_copy(rows_v, agg_s.at[dst_v.at[0]], add=True)
        return carry
    lax.fori_loop(0, NCHUNKS, _chunk, 0)
    plsc.subcore_barrier()

    for off, nr in _ROW_CHUNKS:
        pltpu.sync_copy(agg_s.at[pl.ds(base_r + off, nr)], rows_v.at[pl.ds(0, nr)])
        pltpu.sync_copy(rows_v.at[pl.ds(0, nr)], out_hbm.at[c, pl.ds(base_r + off, nr)])


def _make_sc_layer(layer):
    mesh = plsc.VectorSubcoreMesh(core_axis_name="c", subcore_axis_name="s")
    return pl.kernel(
        functools.partial(_sc_body, layer),
        out_type=jax.ShapeDtypeStruct((NC, NP, H), jnp.float32),
        mesh=mesh,
        scratch_types=[
            pltpu.VMEM((1, CHUNK), jnp.int32),
            pltpu.VMEM((1, CHUNK), jnp.int32),
            pltpu.VMEM((CHUNK, H), jnp.float32),
            pltpu.VMEM((CHUNK, H), jnp.float32),
            pltpu.VMEM_SHARED((NP, H), jnp.float32),
            pltpu.SemaphoreType.DMA,
        ],
    )


def _enc_kernel(x_ref, w_ref, b_ref, o_ref):
    o_ref[...] = jnp.dot(x_ref[...], w_ref[...],
                         preferred_element_type=jnp.float32) + b_ref[...]


def _eemb_kernel(a_ref, w_ref, b_ref, o_ref):
    o_ref[0] = jnp.dot(a_ref[...], w_ref[0],
                       preferred_element_type=jnp.float32) + b_ref[...]


def _mlp_kernel(h_ref, aggp_ref, eps_ref, w1_ref, b1_ref, g1_ref, bb1_ref,
                w2_ref, b2_ref, g2_ref, bb2_ref, o_ref):
    h = h_ref[...]
    agg = aggp_ref[0, :N, :] + aggp_ref[1, :N, :]
    z = (1.0 + eps_ref[0]) * h + agg
    a = jnp.dot(z, w1_ref[...], preferred_element_type=jnp.float32) + b1_ref[...]
    m = jnp.mean(a, axis=0)
    v = jnp.mean((a - m) ** 2, axis=0)
    a = (a - m) / jnp.sqrt(v + 1e-5) * g1_ref[...] + bb1_ref[...]
    a = jnp.maximum(a, 0.0)
    cz = jnp.dot(a, w2_ref[...], preferred_element_type=jnp.float32) + b2_ref[...]
    m2 = jnp.mean(cz, axis=0)
    v2 = jnp.mean((cz - m2) ** 2, axis=0)
    cz = (cz - m2) / jnp.sqrt(v2 + 1e-5) * g2_ref[...] + bb2_ref[...]
    o_ref[...] = jnp.maximum(cz, 0.0)


def _pool_kernel(h_ref, batch_ref, o_ref):
    b2 = batch_ref[...]
    gids = lax.broadcasted_iota(jnp.int32, (N, G), 1)
    oh = (b2 == gids).astype(jnp.float32)
    sums = lax.dot_general(oh, h_ref[...], (((0,), (0,)), ((), ())),
                           preferred_element_type=jnp.float32)
    counts = jnp.sum(oh, axis=0)
    o_ref[...] = sums / jnp.maximum(counts, 1.0)[:, None]


def kernel(x, edge_index, edge_attr, batch, enc_W, enc_b, edge_W, edge_b, eps,
           mlp1_W, mlp1_b, mlpbn_g, mlpbn_b, mlp2_W, mlp2_b, bn_g, bn_b):
    src = edge_index[0]
    dst = edge_index[1]
    pad = EP - E
    src2 = jnp.concatenate([src, jnp.zeros((pad,), jnp.int32)]).reshape(EP // CHUNK, CHUNK)
    dst2 = jnp.concatenate([dst, jnp.full((pad,), N, jnp.int32)]).reshape(EP // CHUNK, CHUNK)
    attr_p = jnp.concatenate([edge_attr, jnp.zeros((pad, DE), jnp.float32)])

    h = pl.pallas_call(
        _enc_kernel,
        out_shape=jax.ShapeDtypeStruct((N, H), jnp.float32),
    )(x, enc_W, enc_b)

    BE = 4096
    eemb = pl.pallas_call(
        _eemb_kernel,
        grid=(L, EP // BE),
        in_specs=[
            pl.BlockSpec((BE, DE), lambda l, b: (b, 0)),
            pl.BlockSpec((1, DE, H), lambda l, b: (l, 0, 0)),
            pl.BlockSpec((1, H), lambda l, b: (l, 0)),
        ],
        out_specs=pl.BlockSpec((1, BE, H), lambda l, b: (l, b, 0)),
        out_shape=jax.ShapeDtypeStruct((L, EP, H), jnp.float32),
    )(attr_p, edge_W, edge_b)

    mlp_call = pl.pallas_call(
        _mlp_kernel,
        in_specs=[pl.BlockSpec(memory_space=pltpu.VMEM)] * 2
        + [pl.BlockSpec(memory_space=pltpu.SMEM)]
        + [pl.BlockSpec(memory_space=pltpu.VMEM)] * 8,
        out_shape=jax.ShapeDtypeStruct((N, H), jnp.float32),
    )

    for i in range(L):
        aggp = _make_sc_layer(i)(h, src2, dst2, eemb)
        h = mlp_call(h, aggp, eps[i].reshape(1), mlp1_W[i], mlp1_b[i],
                     mlpbn_g[i], mlpbn_b[i], mlp2_W[i], mlp2_b[i],
                     bn_g[i], bn_b[i])

    return pl.pallas_call(
        _pool_kernel,
        out_shape=jax.ShapeDtypeStruct((G, H), jnp.float32),
    )(h, batch.reshape(N, 1))


# speedup vs baseline: 1.6660x; 1.6660x over previous
import functools

import jax
import jax.numpy as jnp
from jax import lax
from jax.experimental import pallas as pl
from jax.experimental.pallas import tpu as pltpu
from jax.experimental.pallas import tpu_sc as plsc

N = 10000
E = 320000
DF = 128
DE = 16
H = 64
L = 3
G = 64

NC = 2
NS = 16
NW = NC * NS
CHUNK = 128
EPW = 10240
EP = NW * EPW
NCHUNKS = EPW // CHUNK
NP = 10112
RPT = NP // NS
_ROW_CHUNKS = ((0, 128), (128, 128), (256, 128), (384, 128), (512, 120))


def _sc_body(layer, h_hbm, src2_hbm, dst2_hbm, eemb_hbm, out_hbm,
             src_v, dst_v, rows_v, emb_v, agg_s, sem):
    c = lax.axis_index("c")
    s = lax.axis_index("s")
    w = s * NC + c

    def _zrow(i, carry):
        for k in range(4):
            rows_v[i, pl.ds(16 * k, 16)] = jnp.zeros((16,), jnp.float32)
        return carry
    lax.fori_loop(0, CHUNK, _zrow, 0, unroll=4)
    base_r = s * RPT
    for off, nr in _ROW_CHUNKS:
        pltpu.sync_copy(rows_v.at[pl.ds(0, nr)], agg_s.at[pl.ds(base_r + off, nr)])
    plsc.subcore_barrier()

    def _chunk(ci, carry):
        r = w * NCHUNKS + ci
        e0 = r * CHUNK
        pltpu.sync_copy(src2_hbm.at[r], src_v.at[0])
        pltpu.sync_copy(dst2_hbm.at[r], dst_v.at[0])
        pltpu.async_copy(h_hbm.at[src_v.at[0]], rows_v, sem).wait()
        pltpu.sync_copy(eemb_hbm.at[layer, pl.ds(e0, CHUNK)], emb_v)

        def _vrow(i, cy):
            for k in range(4):
                sl = pl.ds(16 * k, 16)
                rows_v[i, sl] = jnp.maximum(rows_v[i, sl] + emb_v[i, sl], 0.0)
            return cy
        lax.fori_loop(0, CHUNK, _vrow, 0, unroll=2)
        pltpu.sync_copy(rows_v, agg_s.at[dst_v.at[0]], add=True)
        return carry
    lax.fori_loop(0, NCHUNKS, _chunk, 0)
    plsc.subcore_barrier()

    for off, nr in _ROW_CHUNKS:
        pltpu.sync_copy(agg_s.at[pl.ds(base_r + off, nr)], rows_v.at[pl.ds(0, nr)])
        pltpu.sync_copy(rows_v.at[pl.ds(0, nr)], out_hbm.at[c, pl.ds(base_r + off, nr)])


def _make_sc_layer(layer):
    mesh = plsc.VectorSubcoreMesh(core_axis_name="c", subcore_axis_name="s")
    return pl.kernel(
        functools.partial(_sc_body, layer),
        out_type=jax.ShapeDtypeStruct((NC, NP, H), jnp.float32),
        mesh=mesh,
        scratch_types=[
            pltpu.VMEM((1, CHUNK), jnp.int32),
            pltpu.VMEM((1, CHUNK), jnp.int32),
            pltpu.VMEM((CHUNK, H), jnp.float32),
            pltpu.VMEM((CHUNK, H), jnp.float32),
            pltpu.VMEM_SHARED((NP, H), jnp.float32),
            pltpu.SemaphoreType.DMA,
        ],
        compiler_params=pltpu.CompilerParams(use_tc_tiling_on_sc=False),
    )


def _enc_kernel(x_ref, w_ref, b_ref, o_ref):
    o_ref[...] = jnp.dot(x_ref[...], w_ref[...],
                         preferred_element_type=jnp.float32) + b_ref[...]


def _eemb_kernel(a_ref, w_ref, b_ref, o_ref):
    o_ref[0] = jnp.dot(a_ref[...], w_ref[0],
                       preferred_element_type=jnp.float32) + b_ref[0]


def _mlp_kernel(h_ref, aggp_ref, eps_ref, w1_ref, b1_ref, g1_ref, bb1_ref,
                w2_ref, b2_ref, g2_ref, bb2_ref, o_ref):
    h = h_ref[...]
    agg = aggp_ref[0, :N, :] + aggp_ref[1, :N, :]
    z = (1.0 + eps_ref[0]) * h + agg
    a = jnp.dot(z, w1_ref[...], preferred_element_type=jnp.float32) + b1_ref[...]
    m = jnp.mean(a, axis=0)
    v = jnp.mean((a - m) ** 2, axis=0)
    a = (a - m) / jnp.sqrt(v + 1e-5) * g1_ref[...] + bb1_ref[...]
    a = jnp.maximum(a, 0.0)
    cz = jnp.dot(a, w2_ref[...], preferred_element_type=jnp.float32) + b2_ref[...]
    m2 = jnp.mean(cz, axis=0)
    v2 = jnp.mean((cz - m2) ** 2, axis=0)
    cz = (cz - m2) / jnp.sqrt(v2 + 1e-5) * g2_ref[...] + bb2_ref[...]
    o_ref[...] = jnp.maximum(cz, 0.0)


def _pool_kernel(h_ref, batch_ref, o_ref):
    b2 = batch_ref[...]
    gids = lax.broadcasted_iota(jnp.int32, (N, G), 1)
    oh = (b2 == gids).astype(jnp.float32)
    sums = lax.dot_general(oh, h_ref[...], (((0,), (0,)), ((), ())),
                           preferred_element_type=jnp.float32)
    counts = jnp.sum(oh, axis=0)
    o_ref[...] = sums / jnp.maximum(counts, 1.0)[:, None]


def kernel(x, edge_index, edge_attr, batch, enc_W, enc_b, edge_W, edge_b, eps,
           mlp1_W, mlp1_b, mlpbn_g, mlpbn_b, mlp2_W, mlp2_b, bn_g, bn_b):
    src = edge_index[0]
    dst = edge_index[1]
    pad = EP - E
    src2 = jnp.concatenate([src, jnp.zeros((pad,), jnp.int32)]).reshape(EP // CHUNK, CHUNK)
    dst2 = jnp.concatenate([dst, jnp.full((pad,), N, jnp.int32)]).reshape(EP // CHUNK, CHUNK)
    attr_p = jnp.concatenate([edge_attr, jnp.zeros((pad, DE), jnp.float32)])

    h = pl.pallas_call(
        _enc_kernel,
        out_shape=jax.ShapeDtypeStruct((N, H), jnp.float32),
    )(x, enc_W, enc_b)

    BE = 4096
    eemb = pl.pallas_call(
        _eemb_kernel,
        grid=(L, EP // BE),
        in_specs=[
            pl.BlockSpec((BE, DE), lambda l, b: (b, 0)),
            pl.BlockSpec((1, DE, H), lambda l, b: (l, 0, 0)),
            pl.BlockSpec((1, 1, H), lambda l, b: (l, 0, 0)),
        ],
        out_specs=pl.BlockSpec((1, BE, H), lambda l, b: (l, b, 0)),
        out_shape=jax.ShapeDtypeStruct((L, EP, H), jnp.float32),
    )(attr_p, edge_W, edge_b.reshape(L, 1, H))

    mlp_call = pl.pallas_call(
        _mlp_kernel,
        in_specs=[pl.BlockSpec(memory_space=pltpu.VMEM)] * 2
        + [pl.BlockSpec(memory_space=pltpu.SMEM)]
        + [pl.BlockSpec(memory_space=pltpu.VMEM)] * 8,
        out_shape=jax.ShapeDtypeStruct((N, H), jnp.float32),
    )

    for i in range(L):
        aggp = _make_sc_layer(i)(h, src2, dst2, eemb)
        h = mlp_call(h, aggp, eps[i].reshape(1), mlp1_W[i], mlp1_b[i],
                     mlpbn_g[i], mlpbn_b[i], mlp2_W[i], mlp2_b[i],
                     bn_g[i], bn_b[i])

    return pl.pallas_call(
        _pool_kernel,
        out_shape=jax.ShapeDtypeStruct((G, H), jnp.float32),
    )(h, batch.reshape(N, 1))


# speedup vs baseline: 2.4920x; 1.4958x over previous
import functools

import jax
import jax.numpy as jnp
from jax import lax
from jax.experimental import pallas as pl
from jax.experimental.pallas import tpu as pltpu
from jax.experimental.pallas import tpu_sc as plsc

N = 10000
E = 320000
DF = 128
DE = 16
H = 64
L = 3
G = 64

NC = 2
NS = 16
NW = NC * NS
CHUNK = 128
EPW = 10240
EP = NW * EPW
NCHUNKS = EPW // CHUNK
NP = 10112
RPT = NP // NS
_ROW_CHUNKS = ((0, 128), (128, 128), (256, 128), (384, 128), (512, 120))


SUP = 2
SE = SUP * CHUNK
NSTEPS = EPW // SE


def _sc_body(layer, h_hbm, src2_hbm, dst2_hbm, eemb_hbm, out_hbm,
             src_all, dst_all, rows0, rows1, emb0, emb1, agg_s,
             semg0, semg1, seme0, seme1):
    c = lax.axis_index("c")
    s = lax.axis_index("s")
    w = s * NC + c
    rows_ = (rows0, rows1)
    emb_ = (emb0, emb1)
    semg_ = (semg0, semg1)
    seme_ = (seme0, seme1)

    pltpu.sync_copy(src2_hbm.at[pl.ds(w * NCHUNKS, NCHUNKS)], src_all)
    pltpu.sync_copy(dst2_hbm.at[pl.ds(w * NCHUNKS, NCHUNKS)], dst_all)

    def _zrow(i, carry):
        for k in range(4):
            rows0[i, pl.ds(16 * k, 16)] = jnp.zeros((16,), jnp.float32)
        return carry
    lax.fori_loop(0, CHUNK, _zrow, 0, unroll=4)
    base_r = s * RPT
    for off, nr in _ROW_CHUNKS:
        pltpu.sync_copy(rows0.at[pl.ds(0, nr)], agg_s.at[pl.ds(base_r + off, nr)])
    plsc.subcore_barrier()

    def _start_fetch(b, i):
        for s_ in range(SUP):
            pltpu.async_copy(h_hbm.at[src_all.at[SUP * i + s_]],
                             rows_[b].at[pl.ds(CHUNK * s_, CHUNK)], semg_[b])
        e0 = (w * NCHUNKS + SUP * i) * CHUNK
        pltpu.async_copy(eemb_hbm.at[layer, pl.ds(e0, SE)], emb_[b], seme_[b])

    def _wait_fetch(b):
        for s_ in range(SUP):
            pltpu.make_async_copy(h_hbm.at[src_all.at[s_]],
                                  rows_[b].at[pl.ds(CHUNK * s_, CHUNK)],
                                  semg_[b]).wait()
        pltpu.make_async_copy(eemb_hbm.at[layer, pl.ds(0, SE)], emb_[b],
                              seme_[b]).wait()

    _start_fetch(0, jnp.int32(0))

    def _outer(g, carry):
        for b in range(2):
            i = 2 * g + b
            _wait_fetch(b)
            inext = jnp.minimum(i + 1, NSTEPS - 1)
            _start_fetch(1 - b, inext)

            def _vrow(j, cy):
                for k in range(4):
                    sl = pl.ds(16 * k, 16)
                    rows_[b][j, sl] = jnp.maximum(rows_[b][j, sl] + emb_[b][j, sl], 0.0)
                return cy
            lax.fori_loop(0, SE, _vrow, 0, unroll=2)
            for s_ in range(SUP):
                pltpu.sync_copy(rows_[b].at[pl.ds(CHUNK * s_, CHUNK)],
                                agg_s.at[dst_all.at[SUP * i + s_]], add=True)
        return carry
    lax.fori_loop(0, NSTEPS // 2, _outer, 0)
    _wait_fetch(0)
    plsc.subcore_barrier()

    for off, nr in _ROW_CHUNKS:
        pltpu.sync_copy(agg_s.at[pl.ds(base_r + off, nr)], rows0.at[pl.ds(0, nr)])
        pltpu.sync_copy(rows0.at[pl.ds(0, nr)], out_hbm.at[c, pl.ds(base_r + off, nr)])


def _make_sc_layer(layer):
    mesh = plsc.VectorSubcoreMesh(core_axis_name="c", subcore_axis_name="s")
    return pl.kernel(
        functools.partial(_sc_body, layer),
        out_type=jax.ShapeDtypeStruct((NC, NP, H), jnp.float32),
        mesh=mesh,
        scratch_types=[
            pltpu.VMEM((NCHUNKS, CHUNK), jnp.int32),
            pltpu.VMEM((NCHUNKS, CHUNK), jnp.int32),
            pltpu.VMEM((SE, H), jnp.float32),
            pltpu.VMEM((SE, H), jnp.float32),
            pltpu.VMEM((SE, H), jnp.float32),
            pltpu.VMEM((SE, H), jnp.float32),
            pltpu.VMEM_SHARED((NP, H), jnp.float32),
            pltpu.SemaphoreType.DMA,
            pltpu.SemaphoreType.DMA,
            pltpu.SemaphoreType.DMA,
            pltpu.SemaphoreType.DMA,
        ],
        compiler_params=pltpu.CompilerParams(use_tc_tiling_on_sc=False),
    )


def _enc_kernel(x_ref, w_ref, b_ref, o_ref):
    o_ref[...] = jnp.dot(x_ref[...], w_ref[...],
                         preferred_element_type=jnp.float32) + b_ref[...]


def _eemb_kernel(a_ref, w_ref, b_ref, o_ref):
    o_ref[0] = jnp.dot(a_ref[...], w_ref[0],
                       preferred_element_type=jnp.float32) + b_ref[0]


def _mlp_kernel(h_ref, aggp_ref, eps_ref, w1_ref, b1_ref, g1_ref, bb1_ref,
                w2_ref, b2_ref, g2_ref, bb2_ref, o_ref):
    h = h_ref[...]
    agg = aggp_ref[0, :N, :] + aggp_ref[1, :N, :]
    z = (1.0 + eps_ref[0]) * h + agg
    a = jnp.dot(z, w1_ref[...], preferred_element_type=jnp.float32) + b1_ref[...]
    m = jnp.mean(a, axis=0)
    v = jnp.mean((a - m) ** 2, axis=0)
    a = (a - m) / jnp.sqrt(v + 1e-5) * g1_ref[...] + bb1_ref[...]
    a = jnp.maximum(a, 0.0)
    cz = jnp.dot(a, w2_ref[...], preferred_element_type=jnp.float32) + b2_ref[...]
    m2 = jnp.mean(cz, axis=0)
    v2 = jnp.mean((cz - m2) ** 2, axis=0)
    cz = (cz - m2) / jnp.sqrt(v2 + 1e-5) * g2_ref[...] + bb2_ref[...]
    o_ref[...] = jnp.maximum(cz, 0.0)


def _pool_kernel(h_ref, batch_ref, o_ref):
    b2 = batch_ref[...]
    gids = lax.broadcasted_iota(jnp.int32, (N, G), 1)
    oh = (b2 == gids).astype(jnp.float32)
    sums = lax.dot_general(oh, h_ref[...], (((0,), (0,)), ((), ())),
                           preferred_element_type=jnp.float32)
    counts = jnp.sum(oh, axis=0)
    o_ref[...] = sums / jnp.maximum(counts, 1.0)[:, None]


def kernel(x, edge_index, edge_attr, batch, enc_W, enc_b, edge_W, edge_b, eps,
           mlp1_W, mlp1_b, mlpbn_g, mlpbn_b, mlp2_W, mlp2_b, bn_g, bn_b):
    src = edge_index[0]
    dst = edge_index[1]
    pad = EP - E
    src2 = jnp.concatenate([src, jnp.zeros((pad,), jnp.int32)]).reshape(EP // CHUNK, CHUNK)
    dst2 = jnp.concatenate([dst, jnp.full((pad,), N, jnp.int32)]).reshape(EP // CHUNK, CHUNK)
    attr_p = jnp.concatenate([edge_attr, jnp.zeros((pad, DE), jnp.float32)])

    h = pl.pallas_call(
        _enc_kernel,
        out_shape=jax.ShapeDtypeStruct((N, H), jnp.float32),
    )(x, enc_W, enc_b)

    BE = 4096
    eemb = pl.pallas_call(
        _eemb_kernel,
        grid=(L, EP // BE),
        in_specs=[
            pl.BlockSpec((BE, DE), lambda l, b: (b, 0)),
            pl.BlockSpec((1, DE, H), lambda l, b: (l, 0, 0)),
            pl.BlockSpec((1, 1, H), lambda l, b: (l, 0, 0)),
        ],
        out_specs=pl.BlockSpec((1, BE, H), lambda l, b: (l, b, 0)),
        out_shape=jax.ShapeDtypeStruct((L, EP, H), jnp.float32),
    )(attr_p, edge_W, edge_b.reshape(L, 1, H))

    mlp_call = pl.pallas_call(
        _mlp_kernel,
        in_specs=[pl.BlockSpec(memory_space=pltpu.VMEM)] * 2
        + [pl.BlockSpec(memory_space=pltpu.SMEM)]
        + [pl.BlockSpec(memory_space=pltpu.VMEM)] * 8,
        out_shape=jax.ShapeDtypeStruct((N, H), jnp.float32),
    )

    for i in range(L):
        aggp = _make_sc_layer(i)(h, src2, dst2, eemb)
        h = mlp_call(h, aggp, eps[i].reshape(1), mlp1_W[i], mlp1_b[i],
                     mlpbn_g[i], mlpbn_b[i], mlp2_W[i], mlp2_b[i],
                     bn_g[i], bn_b[i])

    return pl.pallas_call(
        _pool_kernel,
        out_shape=jax.ShapeDtypeStruct((G, H), jnp.float32),
    )(h, batch.reshape(N, 1))
